# baseline (speedup 1.0000x reference)
import functools

import numpy as np
import jax
import jax.numpy as jnp
from jax import lax
from jax.experimental import pallas as pl
from jax.experimental.pallas import tpu as pltpu
from jax.experimental.pallas import tpu_sc as plsc

B = 4096
S = 26
D = 64
NT = S + 1
H0, H1, H2 = 512, 256, 64
T0, T1 = 512, 256
NW = 32
ROWS = B * S
CHUNK = 128
CPW = ROWS // (NW * CHUNK)
PW = ROWS // NW

_sc_mesh = plsc.VectorSubcoreMesh(core_axis_name="c", subcore_axis_name="s")


@functools.partial(
    pl.kernel,
    mesh=_sc_mesh,
    out_type=jax.ShapeDtypeStruct((ROWS, D), jnp.float32),
    scratch_types=[
        pltpu.VMEM((CPW, CHUNK), jnp.int32),
        pltpu.VMEM((CHUNK, D), jnp.float32),
        pltpu.SemaphoreType.DMA,
    ],
    compiler_params=pltpu.CompilerParams(use_tc_tiling_on_sc=False),
)
def _sc_gather(table_hbm, idx_hbm, out_hbm, idx_v, rows_v, sem):
    wid = lax.axis_index("s") * 2 + lax.axis_index("c")
    pltpu.sync_copy(idx_hbm.at[wid], idx_v)
    base = wid * PW

    def body(j, carry):
        pltpu.async_copy(table_hbm.at[idx_v.at[j]], rows_v, sem).wait()
        pltpu.sync_copy(rows_v, out_hbm.at[pl.ds(base + j * CHUNK, CHUNK)])
        return carry

    lax.fori_loop(0, CPW, body, 0)


BLK = 512


def _bot_body(dx, w0, b0, w1, b1, w2, b2, out):
    x = jnp.maximum(dx[...] @ w0[...] + b0[...], 0.0)
    x = jnp.maximum(x @ w1[...] + b1[...], 0.0)
    out[...] = jnp.maximum(x @ w2[...] + b2[...], 0.0)


def _bot_call(dx, w0, b0, w1, b1, w2, b2):
    full = lambda shape: pl.BlockSpec(shape, lambda i: (0, 0))
    return pl.pallas_call(
        _bot_body,
        grid=(B // BLK,),
        in_specs=[
            pl.BlockSpec((BLK, 13), lambda i: (i, 0)),
            full((13, H0)), full((1, H0)),
            full((H0, H1)), full((1, H1)),
            full((H1, H2)), full((1, H2)),
        ],
        out_specs=pl.BlockSpec((BLK, H2), lambda i: (i, 0)),
        out_shape=jax.ShapeDtypeStruct((B, H2), jnp.float32),
    )(dx, w0, b0, w1, b1, w2, b2)


def _top_body(x64, emb, w0a, wp, tb0, tw1, tb1, tw2, tb2, out):
    x = x64[...]
    E = emb[...]
    acc = x @ w0a[...] + tb0[...]
    for m in range(S):
        tm = x.reshape(BLK, 1, D) if m == 0 else E[:, m - 1:m, :]
        z = jnp.sum(E * tm, axis=2)
        acc = acc + z @ wp[m]
    y = jnp.maximum(acc, 0.0)
    y = jnp.maximum(y @ tw1[...] + tb1[...], 0.0)
    out[...] = jax.nn.sigmoid(y @ tw2[...] + tb2[...])


def _top_call(x64, emb3, w0a, wp, tb0, tw1, tb1, tw2, tb2):
    full = lambda shape: pl.BlockSpec(shape, lambda i: tuple(0 for _ in shape))
    return pl.pallas_call(
        _top_body,
        grid=(B // BLK,),
        in_specs=[
            pl.BlockSpec((BLK, D), lambda i: (i, 0)),
            pl.BlockSpec((BLK, S, D), lambda i: (i, 0, 0)),
            full((D, T0)),
            full((S, S, T0)),
            full((1, T0)),
            full((T0, T1)), full((1, T1)),
            full((T1, 1)), full((1, 1)),
        ],
        out_specs=pl.BlockSpec((BLK, 1), lambda i: (i, 0)),
        out_shape=jax.ShapeDtypeStruct((B, 1), jnp.float32),
    )(x64, emb3, w0a, wp, tb0, tw1, tb1, tw2, tb2)


_NI, _NJ = np.tril_indices(NT, -1)


def kernel(dense_x, emb_table, bot_W0, bot_b0, bot_W1, bot_b1, bot_W2, bot_b2,
           top_W0, top_b0, top_W1, top_b1, top_W2, top_b2, sparse_idx):
    idx2 = sparse_idx.astype(jnp.int32).reshape(NW, CPW, CHUNK)
    emb_flat = _sc_gather(emb_table, idx2)
    emb3 = emb_flat.reshape(B, S, D)

    x64 = _bot_call(dense_x, bot_W0.T, bot_b0.reshape(1, -1),
                    bot_W1.T, bot_b1.reshape(1, -1),
                    bot_W2.T, bot_b2.reshape(1, -1))

    wp = jnp.zeros((S, S, T0), jnp.float32).at[_NJ, _NI - 1, :].set(top_W0[:, D:].T)
    w0a = top_W0[:, :D].T

    return _top_call(x64, emb3, w0a, wp, top_b0.reshape(1, -1),
                     top_W1.T, top_b1.reshape(1, -1),
                     top_W2.T, top_b2.reshape(1, -1))
